# Initial kernel scaffold; baseline (speedup 1.0000x reference)
import jax
import jax.numpy as jnp
from jax.experimental import pallas as pl


def kernel(node_embeddings, initial_features, graph_nodes_list, num_graphs, Wg, bg, Wt, bt, Wo, bo):
    raise NotImplementedError("write your pallas kernel here")


# speedup vs baseline: 3.4948x; 3.4948x over previous
import functools

import jax
import jax.numpy as jnp
from jax import lax
from jax.experimental import pallas as pl
from jax.experimental.pallas import tpu as pltpu
from jax.experimental.pallas import tpu_sc as plsc

N = 320000
H = 128
G = 1024

_BLK = 2000


def _gate_body(emb_ref, feat_ref, wg1_ref, wg2_ref, bg_ref, wt_ref, bt_ref,
               out_ref):
    emb = emb_ref[...]
    feat = feat_ref[...]
    z = (jnp.dot(emb, wg1_ref[...], preferred_element_type=jnp.float32)
         + jnp.dot(feat, wg2_ref[...], preferred_element_type=jnp.float32)
         + bg_ref[...])
    gate = jax.nn.sigmoid(z)
    t = jnp.tanh(
        jnp.dot(emb, wt_ref[...], preferred_element_type=jnp.float32)
        + bt_ref[...])
    out_ref[...] = gate * t


def _gate_call(emb, feat, wg1t, wg2t, bg2, wtt, bt2):
    grid = (N // _BLK,)
    row_spec = pl.BlockSpec((_BLK, H), lambda i: (i, 0))
    w_spec = pl.BlockSpec((H, H), lambda i: (0, 0))
    b_spec = pl.BlockSpec((1, H), lambda i: (0, 0))
    return pl.pallas_call(
        _gate_body,
        grid=grid,
        in_specs=[row_spec, row_spec, w_spec, w_spec, b_spec, w_spec, b_spec],
        out_specs=row_spec,
        out_shape=jax.ShapeDtypeStruct((N, H), jnp.float32),
    )(emb, feat, wg1t, wg2t, bg2, wtt, bt2)


_R = N // H
_RP = 2504
_SC_TOTAL = _RP // 8
_SC_FULL = N // 1024
_GPS = G // 16


def _segsum_body(gated, ids, zeros64, out, idx_v, rows_v, stage_v, acc):
    c = lax.axis_index("c")
    s = lax.axis_index("s")
    wid = s * 2 + c

    pltpu.sync_copy(zeros64, stage_v)
    pltpu.sync_copy(stage_v, acc.at[pl.ds(s * _GPS, _GPS)])
    plsc.subcore_barrier()

    base = jnp.where(wid < 25, wid * 10, 250 + (wid - 25) * 9)
    cnt = jnp.where(wid < 25, 10, 9)

    def chunk_body(j, carry):
        sc = base + j
        row0 = sc * 8
        node0 = sc * 1024
        pltpu.sync_copy(ids.at[pl.ds(row0, 8)], idx_v)
        pltpu.sync_copy(gated.at[pl.ds(node0, 512)], rows_v)
        for q in range(4):
            pltpu.sync_copy(rows_v.at[pl.ds(q * H, H)], acc.at[idx_v.at[q]],
                            add=True)

        @pl.when(sc < _SC_FULL)
        def _second_half():
            pltpu.sync_copy(gated.at[pl.ds(node0 + 512, 512)], rows_v)
            for q in range(4):
                pltpu.sync_copy(rows_v.at[pl.ds(q * H, H)],
                                acc.at[idx_v.at[4 + q]], add=True)

        return carry

    lax.fori_loop(0, cnt, chunk_body, 0)
    plsc.subcore_barrier()

    pltpu.sync_copy(acc.at[pl.ds(s * _GPS, _GPS)], stage_v)
    pltpu.sync_copy(stage_v, out.at[c].at[pl.ds(s * _GPS, _GPS)])


_segsum_call = functools.partial(
    pl.kernel,
    out_type=jax.ShapeDtypeStruct((2, G, H), jnp.float32),
    mesh=plsc.VectorSubcoreMesh(core_axis_name="c", subcore_axis_name="s"),
    scratch_types=[
        pltpu.VMEM((8, H), jnp.int32),
        pltpu.VMEM((512, H), jnp.float32),
        pltpu.VMEM((_GPS, H), jnp.float32),
        pltpu.VMEM_SHARED((G, H), jnp.float32),
    ],
)(_segsum_body)


def _combine_body(p_ref, wo_ref, bo_ref, pred_ref, repr_ref):
    grepr = p_ref[0] + p_ref[1]
    repr_ref[...] = grepr
    pred_ref[...] = (jnp.sum(grepr * wo_ref[...], axis=1, keepdims=True)
                     + bo_ref[...])


def _combine_call(partials, wo, bo2):
    return pl.pallas_call(
        _combine_body,
        out_shape=(
            jax.ShapeDtypeStruct((G, 1), jnp.float32),
            jax.ShapeDtypeStruct((G, H), jnp.float32),
        ),
    )(partials, wo, bo2)


def kernel(node_embeddings, initial_features, graph_nodes_list, num_graphs,
           Wg, bg, Wt, bt, Wo, bo):
    wg1t = Wg[:, :H].T
    wg2t = Wg[:, H:].T
    wtt = Wt.T
    gated = _gate_call(node_embeddings, initial_features, wg1t, wg2t,
                       bg.reshape(1, H), wtt, bt.reshape(1, H))
    ids2d = jnp.concatenate(
        [graph_nodes_list,
         jnp.zeros((_RP * H - N,), jnp.int32)]).reshape(_RP, H)
    zeros64 = jnp.zeros((_GPS, H), jnp.float32)
    partials = _segsum_call(gated, ids2d, zeros64)
    pred, graph_repr = _combine_call(partials, Wo, bo.reshape(1, 1))
    return pred.reshape(G), graph_repr


# speedup vs baseline: 3.8663x; 1.1063x over previous
import functools

import jax
import jax.numpy as jnp
from jax import lax
from jax.experimental import pallas as pl
from jax.experimental.pallas import tpu as pltpu
from jax.experimental.pallas import tpu_sc as plsc

N = 320000
H = 128
G = 1024

_BLK = 2000


def _gate_body(emb_ref, feat_ref, wg1_ref, wg2_ref, bg_ref, wt_ref, bt_ref,
               out_ref):
    emb = emb_ref[...]
    feat = feat_ref[...]
    z = (jnp.dot(emb, wg1_ref[...], preferred_element_type=jnp.float32)
         + jnp.dot(feat, wg2_ref[...], preferred_element_type=jnp.float32)
         + bg_ref[...])
    gate = jax.nn.sigmoid(z)
    t = jnp.tanh(
        jnp.dot(emb, wt_ref[...], preferred_element_type=jnp.float32)
        + bt_ref[...])
    out_ref[...] = gate * t


def _gate_call(emb, feat, wg1t, wg2t, bg2, wtt, bt2):
    grid = (N // _BLK,)
    row_spec = pl.BlockSpec((_BLK, H), lambda i: (i, 0))
    w_spec = pl.BlockSpec((H, H), lambda i: (0, 0))
    b_spec = pl.BlockSpec((1, H), lambda i: (0, 0))
    return pl.pallas_call(
        _gate_body,
        grid=grid,
        in_specs=[row_spec, row_spec, w_spec, w_spec, b_spec, w_spec, b_spec],
        out_specs=row_spec,
        out_shape=jax.ShapeDtypeStruct((N, H), jnp.float32),
    )(emb, feat, wg1t, wg2t, bg2, wtt, bt2)


_R = N // H
_RP = 2504
_SC_TOTAL = _RP // 8
_SC_FULL = N // 1024
_GPS = G // 16


_Q = 256


def _segsum_body(gated, ids, zeros64, out, idx_v, rows0, rows1, stage_v, acc,
                 sem0, sem1):
    c = lax.axis_index("c")
    s = lax.axis_index("s")
    wid = s * 2 + c
    bufs = (rows0, rows1)
    sems = (sem0, sem1)

    pltpu.sync_copy(zeros64, stage_v)
    pltpu.sync_copy(stage_v, acc.at[pl.ds(s * _GPS, _GPS)])
    plsc.subcore_barrier()

    base = jnp.where(wid < 25, wid * 10, 250 + (wid - 25) * 9)
    cnt = jnp.where(wid < 25, 10, jnp.where(wid < 31, 9, 8))

    def gather_start(node0, buf, sem):
        pltpu.make_async_copy(gated.at[pl.ds(node0, _Q)], buf, sem).start()

    def gather_wait(buf, sem):
        pltpu.make_async_copy(gated.at[pl.ds(0, _Q)], buf, sem).wait()

    gather_start(base * 1024, rows0, sem0)

    def chunk_body(j, carry):
        sc = base + j
        node0 = sc * 1024
        pltpu.sync_copy(ids.at[pl.ds(sc * 8, 8)], idx_v)
        for q in range(4):
            buf, sem = bufs[q % 2], sems[q % 2]
            nbuf, nsem = bufs[(q + 1) % 2], sems[(q + 1) % 2]
            gather_wait(buf, sem)
            if q < 3:
                gather_start(node0 + (q + 1) * _Q, nbuf, nsem)
            else:
                @pl.when(j + 1 < cnt)
                def _prefetch_next():
                    gather_start(node0 + 1024, nbuf, nsem)
            for h in range(2):
                pltpu.sync_copy(buf.at[pl.ds(h * H, H)],
                                acc.at[idx_v.at[2 * q + h]], add=True)
        return carry

    lax.fori_loop(0, cnt, chunk_body, 0)

    @pl.when(wid == 31)
    def _tail():
        pltpu.sync_copy(ids.at[pl.ds(_SC_FULL * 8, 8)], idx_v)
        for q in range(2):
            pltpu.sync_copy(gated.at[pl.ds(_SC_FULL * 1024 + q * _Q, _Q)],
                            rows0)
            for h in range(2):
                pltpu.sync_copy(rows0.at[pl.ds(h * H, H)],
                                acc.at[idx_v.at[2 * q + h]], add=True)

    plsc.subcore_barrier()

    pltpu.sync_copy(acc.at[pl.ds(s * _GPS, _GPS)], stage_v)
    pltpu.sync_copy(stage_v, out.at[c].at[pl.ds(s * _GPS, _GPS)])


_segsum_call = functools.partial(
    pl.kernel,
    out_type=jax.ShapeDtypeStruct((2, G, H), jnp.float32),
    mesh=plsc.VectorSubcoreMesh(core_axis_name="c", subcore_axis_name="s"),
    scratch_types=[
        pltpu.VMEM((8, H), jnp.int32),
        pltpu.VMEM((_Q, H), jnp.float32),
        pltpu.VMEM((_Q, H), jnp.float32),
        pltpu.VMEM((_GPS, H), jnp.float32),
        pltpu.VMEM_SHARED((G, H), jnp.float32),
        pltpu.SemaphoreType.DMA,
        pltpu.SemaphoreType.DMA,
    ],
)(_segsum_body)


def _combine_body(p_ref, wo_ref, bo_ref, pred_ref, repr_ref):
    grepr = p_ref[0] + p_ref[1]
    repr_ref[...] = grepr
    pred_ref[...] = (jnp.sum(grepr * wo_ref[...], axis=1, keepdims=True)
                     + bo_ref[...])


def _combine_call(partials, wo, bo2):
    return pl.pallas_call(
        _combine_body,
        out_shape=(
            jax.ShapeDtypeStruct((G, 1), jnp.float32),
            jax.ShapeDtypeStruct((G, H), jnp.float32),
        ),
    )(partials, wo, bo2)


def kernel(node_embeddings, initial_features, graph_nodes_list, num_graphs,
           Wg, bg, Wt, bt, Wo, bo):
    wg1t = Wg[:, :H].T
    wg2t = Wg[:, H:].T
    wtt = Wt.T
    gated = _gate_call(node_embeddings, initial_features, wg1t, wg2t,
                       bg.reshape(1, H), wtt, bt.reshape(1, H))
    ids2d = jnp.concatenate(
        [graph_nodes_list,
         jnp.zeros((_RP * H - N,), jnp.int32)]).reshape(_RP, H)
    zeros64 = jnp.zeros((_GPS, H), jnp.float32)
    partials = _segsum_call(gated, ids2d, zeros64)
    pred, graph_repr = _combine_call(partials, Wo, bo.reshape(1, 1))
    return pred.reshape(G), graph_repr
